# Initial kernel scaffold; baseline (speedup 1.0000x reference)
import jax
import jax.numpy as jnp
from jax.experimental import pallas as pl


def kernel(adj, student, teacher, topk):
    raise NotImplementedError("write your pallas kernel here")


# speedup vs baseline: 15.6746x; 15.6746x over previous
import jax
import jax.numpy as jnp
from jax import lax
from jax.experimental import pallas as pl
from jax.experimental.pallas import tpu as pltpu
from jax.experimental.pallas import tpu_sc as plsc

N = 4096
D = 64
K = 16
NCENT = 32
NSEEDS = 5
NITER = 20

ROWS_BLK = 256
NBLK = N // ROWS_BLK


def _topk_body(s_ref, tT_ref, out_ref):
    i = pl.program_id(0)
    sim = lax.dot_general(
        s_ref[...], tT_ref[...], (((1,), (0,)), ((), ())),
        preferred_element_type=jnp.float32)
    rows = i * ROWS_BLK + lax.broadcasted_iota(jnp.int32, (ROWS_BLK, 1), 0)
    cols = lax.broadcasted_iota(jnp.int32, (ROWS_BLK, N), 1)
    work = jnp.where(cols == rows, sim + 10.0, sim)
    neg = jnp.float32(-jnp.inf)
    picked = []
    for _ in range(K):
        m = jnp.max(work, axis=1, keepdims=True)
        idx = jnp.min(jnp.where(work == m, cols, N), axis=1, keepdims=True)
        picked.append(idx)
        work = jnp.where(cols == idx, neg, work)
    k = 2
    while k <= K:
        j = k // 2
        while j >= 1:
            for i in range(K):
                l = i ^ j
                if l > i:
                    a, b = picked[i], picked[l]
                    lo, hi = jnp.minimum(a, b), jnp.maximum(a, b)
                    if (i & k) == 0:
                        picked[i], picked[l] = lo, hi
                    else:
                        picked[i], picked[l] = hi, lo
            j //= 2
        k *= 2
    out_ref[...] = jnp.concatenate(picked, axis=1)


def _topk(student, teacher_t):
    return pl.pallas_call(
        _topk_body,
        grid=(NBLK,),
        in_specs=[
            pl.BlockSpec((ROWS_BLK, D), lambda i: (i, 0)),
            pl.BlockSpec((D, N), lambda i: (0, 0)),
        ],
        out_specs=pl.BlockSpec((ROWS_BLK, K), lambda i: (i, 0)),
        out_shape=jax.ShapeDtypeStruct((N, K), jnp.int32),
    )(student, teacher_t)


def _kmeans_body(x_ref, xT_ref, c0_ref, out_ref):
    x = x_ref[...]
    xT = xT_ref[...]
    xsqT = jnp.sum(xT * xT, axis=0, keepdims=True)
    iota_c = lax.broadcasted_iota(jnp.int32, (NCENT, N), 0)

    def assign(c):
        csq = jnp.sum(c * c, axis=1, keepdims=True)
        prod = lax.dot_general(
            c, xT, (((1,), (0,)), ((), ())),
            preferred_element_type=jnp.float32)
        d2 = xsqT - 2.0 * prod + csq
        m = jnp.min(d2, axis=0, keepdims=True)
        return jnp.min(jnp.where(d2 == m, iota_c, NCENT), axis=0,
                       keepdims=True)

    def step(_, c):
        lbl = assign(c)
        h = (iota_c == lbl).astype(jnp.float32)
        sums = lax.dot_general(
            h, x, (((1,), (0,)), ((), ())),
            preferred_element_type=jnp.float32)
        counts = jnp.sum(h, axis=1, keepdims=True)
        return jnp.where(counts > 0.0,
                         sums / jnp.maximum(counts, 1.0), c)

    c = lax.fori_loop(0, NITER, step, c0_ref[0])
    out_ref[...] = assign(c)[None]


def _kmeans_labels(x, x_t, cent0):
    return pl.pallas_call(
        _kmeans_body,
        grid=(NSEEDS,),
        in_specs=[
            pl.BlockSpec((N, D), lambda s: (0, 0)),
            pl.BlockSpec((D, N), lambda s: (0, 0)),
            pl.BlockSpec((1, NCENT, D), lambda s: (s, 0, 0)),
        ],
        out_specs=pl.BlockSpec((1, 1, N), lambda s: (s, 0, 0)),
        out_shape=jax.ShapeDtypeStruct((NSEEDS, 1, N), jnp.int32),
    )(x, x_t, cent0)


NTILES = 16
RPT = N // NTILES
EPT = RPT * K
GROWS = EPT // 128


def _sc_body(adj_hbm, iknn_hbm, labels_hbm, rows_hbm, cols_hbm,
             iknn_v, labels_v, lk_v, k01_v, klocal_v, dlocal_v,
             gidx_v, avals_v, dest_v, rowv_v, colv_v,
             me_v, call_v, shared, gsem, ssem):
    sid = lax.axis_index("s")
    row0 = sid * RPT
    lanes = lax.iota(jnp.int32, 16)

    pltpu.sync_copy(iknn_hbm.at[pl.ds(sid * GROWS, GROWS)], iknn_v)
    pltpu.sync_copy(labels_hbm, labels_v)

    def pass1(r, carry):
        g = r // 8
        o = (r % 8) * 16
        scols = iknn_v[g, pl.ds(o, 16)]
        rglob = row0 + r
        gidx_v[g, pl.ds(o, 16)] = rglob * N + scols
        keep = jnp.zeros((16,), jnp.int32)
        for s in range(NSEEDS):
            lr = plsc.load_gather(
                labels_v, [jnp.full((16,), s * N, jnp.int32) + rglob])
            ln = plsc.load_gather(labels_v, [scols + s * N])
            keep = keep | (ln == lr).astype(jnp.int32)
        lk_v[pl.ds(r * 16, 16)] = keep
        return carry

    lax.fori_loop(0, RPT, pass1, 0)

    descs = []
    for j in range(GROWS):
        descs.append(
            pltpu.async_copy(adj_hbm.at[gidx_v.at[j]], avals_v.at[j], gsem))
    for d in descs:
        d.wait()

    def pass2(v, carry):
        krun, drun = carry
        g = v // 8
        o = (v % 8) * 16
        av = avals_v[g, pl.ds(o, 16)]
        lk = lk_v[pl.ds(v * 16, 16)]
        k01 = ((av != 0.0) | (lk != 0)).astype(jnp.int32)
        nk = jnp.sum(k01)
        k01_v[pl.ds(v * 16, 16)] = k01
        klocal_v[pl.ds(v * 16, 16)] = krun + plsc.cumsum(k01) - 1
        dlocal_v[pl.ds(v * 16, 16)] = drun + plsc.cumsum(1 - k01) - 1
        return (krun + nk, drun + (16 - nk))

    kcount, _ = lax.fori_loop(0, EPT // 16, pass2,
                              (jnp.int32(0), jnp.int32(0)))

    me_v[...] = jnp.where(lanes == sid, kcount, 0)
    pltpu.sync_copy(me_v, shared.at[sid])
    plsc.subcore_barrier()
    pltpu.sync_copy(shared, call_v)
    counts = jnp.zeros((16,), jnp.int32)
    for t in range(NTILES):
        counts = counts + call_v[t, :]
    total_kept = jnp.sum(counts)
    kbase = jnp.sum(jnp.where(lanes < sid, counts, 0))
    dbase = sid * EPT - kbase

    def pass3(v, carry):
        g = v // 8
        o = (v % 8) * 16
        keep = k01_v[pl.ds(v * 16, 16)] != 0
        kdest = kbase + klocal_v[pl.ds(v * 16, 16)]
        ddest = total_kept + dbase + dlocal_v[pl.ds(v * 16, 16)]
        dest_v[g, pl.ds(o, 16)] = jnp.where(keep, kdest, ddest)
        rowv_v[g, pl.ds(o, 16)] = jnp.where(keep, row0 + v, 0)
        colv_v[g, pl.ds(o, 16)] = jnp.where(keep, iknn_v[g, pl.ds(o, 16)], 0)
        return carry

    lax.fori_loop(0, EPT // 16, pass3, 0)

    descs = []
    for j in range(GROWS):
        descs.append(
            pltpu.async_copy(rowv_v.at[j], rows_hbm.at[dest_v.at[j]], ssem))
        descs.append(
            pltpu.async_copy(colv_v.at[j], cols_hbm.at[dest_v.at[j]], ssem))
    for d in descs:
        d.wait()


def _sc_build(adj_flat, iknn2d, labels_flat):
    mesh = plsc.VectorSubcoreMesh(
        core_axis_name="c", subcore_axis_name="s", num_cores=1)
    f = pl.kernel(
        _sc_body,
        compiler_params=pltpu.CompilerParams(
            use_tc_tiling_on_sc=False, needs_layout_passes=False),
        out_type=(
            jax.ShapeDtypeStruct((N * K,), jnp.int32),
            jax.ShapeDtypeStruct((N * K,), jnp.int32),
        ),
        mesh=mesh,
        scratch_types=[
            pltpu.VMEM((GROWS, 128), jnp.int32),
            pltpu.VMEM((NSEEDS * N,), jnp.int32),
            pltpu.VMEM((EPT,), jnp.int32),
            pltpu.VMEM((EPT,), jnp.int32),
            pltpu.VMEM((EPT,), jnp.int32),
            pltpu.VMEM((EPT,), jnp.int32),
            pltpu.VMEM((GROWS, 128), jnp.int32),
            pltpu.VMEM((GROWS, 128), jnp.float32),
            pltpu.VMEM((GROWS, 128), jnp.int32),
            pltpu.VMEM((GROWS, 128), jnp.int32),
            pltpu.VMEM((GROWS, 128), jnp.int32),
            pltpu.VMEM((16,), jnp.int32),
            pltpu.VMEM((16, 16), jnp.int32),
            pltpu.VMEM_SHARED((16, 16), jnp.int32),
            pltpu.SemaphoreType.DMA,
            pltpu.SemaphoreType.DMA,
        ],
    )
    return f(adj_flat, iknn2d, labels_flat)


def kernel(adj, student, teacher, topk):
    teacher_t = teacher.T
    i_knn = _topk(student, teacher_t)

    cent0 = []
    for s in range(NSEEDS):
        kk = jax.random.key(s + 1234)
        init_idx = jax.random.choice(kk, N, (NCENT,), replace=False)
        cent0.append(teacher[init_idx])
    cent0 = jnp.stack(cent0)
    labels = _kmeans_labels(teacher, teacher_t, cent0)

    rows_out, cols_out = _sc_build(
        adj.reshape(-1), i_knn.reshape(N * K // 128, 128),
        labels.reshape(-1))
    indices = jnp.stack([rows_out, cols_out], axis=0)
    return indices, topk


# speedup vs baseline: 15.7129x; 1.0024x over previous
import jax
import jax.numpy as jnp
from jax import lax
from jax.experimental import pallas as pl
from jax.experimental.pallas import tpu as pltpu
from jax.experimental.pallas import tpu_sc as plsc

N = 4096
D = 64
K = 16
NCENT = 32
NSEEDS = 5
NITER = 20

ROWS_BLK = 256
NBLK = N // ROWS_BLK


def _topk_body(s_ref, tT_ref, out_ref):
    i = pl.program_id(0)
    sim = lax.dot_general(
        s_ref[...], tT_ref[...], (((1,), (0,)), ((), ())),
        preferred_element_type=jnp.float32)
    rows = i * ROWS_BLK + lax.broadcasted_iota(jnp.int32, (ROWS_BLK, 1), 0)
    cols = lax.broadcasted_iota(jnp.int32, (ROWS_BLK, N), 1)
    work = jnp.where(cols == rows, sim + 10.0, sim)
    neg = jnp.float32(-jnp.inf)
    picked = []
    for _ in range(K):
        m = jnp.max(work, axis=1, keepdims=True)
        idx = jnp.min(jnp.where(work == m, cols, N), axis=1, keepdims=True)
        picked.append(idx)
        work = jnp.where(cols == idx, neg, work)
    k = 2
    while k <= K:
        j = k // 2
        while j >= 1:
            for i in range(K):
                l = i ^ j
                if l > i:
                    a, b = picked[i], picked[l]
                    lo, hi = jnp.minimum(a, b), jnp.maximum(a, b)
                    if (i & k) == 0:
                        picked[i], picked[l] = lo, hi
                    else:
                        picked[i], picked[l] = hi, lo
            j //= 2
        k *= 2
    out_ref[...] = jnp.concatenate(picked, axis=1)


def _topk(student, teacher_t):
    return pl.pallas_call(
        _topk_body,
        grid=(NBLK,),
        in_specs=[
            pl.BlockSpec((ROWS_BLK, D), lambda i: (i, 0)),
            pl.BlockSpec((D, N), lambda i: (0, 0)),
        ],
        out_specs=pl.BlockSpec((ROWS_BLK, K), lambda i: (i, 0)),
        out_shape=jax.ShapeDtypeStruct((N, K), jnp.int32),
    )(student, teacher_t)


def _kmeans_body(x_ref, xT_ref, c0_ref, out_ref):
    x = x_ref[...]
    xT = xT_ref[...]
    xsqT = jnp.sum(xT * xT, axis=0, keepdims=True)
    iota_c = lax.broadcasted_iota(jnp.int32, (NCENT, N), 0)

    def assign(c):
        csq = jnp.sum(c * c, axis=1, keepdims=True)
        prod = lax.dot_general(
            c, xT, (((1,), (0,)), ((), ())),
            preferred_element_type=jnp.float32)
        d2 = xsqT - 2.0 * prod + csq
        m = jnp.min(d2, axis=0, keepdims=True)
        return jnp.min(jnp.where(d2 == m, iota_c, NCENT), axis=0,
                       keepdims=True)

    def step(_, c):
        lbl = assign(c)
        h = (iota_c == lbl).astype(jnp.float32)
        sums = lax.dot_general(
            h, x, (((1,), (0,)), ((), ())),
            preferred_element_type=jnp.float32)
        counts = jnp.sum(h, axis=1, keepdims=True)
        return jnp.where(counts > 0.0,
                         sums / jnp.maximum(counts, 1.0), c)

    c = lax.fori_loop(0, NITER, step, c0_ref[0])
    out_ref[...] = assign(c)[None]


def _kmeans_labels(x, x_t, cent0):
    return pl.pallas_call(
        _kmeans_body,
        grid=(NSEEDS,),
        in_specs=[
            pl.BlockSpec((N, D), lambda s: (0, 0)),
            pl.BlockSpec((D, N), lambda s: (0, 0)),
            pl.BlockSpec((1, NCENT, D), lambda s: (s, 0, 0)),
        ],
        out_specs=pl.BlockSpec((1, 1, N), lambda s: (s, 0, 0)),
        out_shape=jax.ShapeDtypeStruct((NSEEDS, 1, N), jnp.int32),
    )(x, x_t, cent0)


NTILES = 16
RPT = N // NTILES
EPT = RPT * K
GROWS = EPT // 128


def _sc_body(adj_hbm, iknn_hbm, labels_hbm, rows_hbm, cols_hbm,
             iknn_v, labels_v, lk_v, k01_v,
             gidx_v, avals_v, dest_v, rowv_v, colv_v,
             me_v, call_v, nkv_s, shared, gsem, ssem):
    sid = lax.axis_index("s")
    row0 = sid * RPT
    lanes = lax.iota(jnp.int32, 16)

    pltpu.sync_copy(iknn_hbm.at[pl.ds(sid * GROWS, GROWS)], iknn_v)
    pltpu.sync_copy(labels_hbm, labels_v)

    @plsc.parallel_loop(0, RPT, 1, unroll=8)
    def pass1a(r):
        g = r >> 3
        o = (r & 7) << 4
        gidx_v[g, pl.ds(o, 16)] = (row0 + r) * N + iknn_v[g, pl.ds(o, 16)]

    descs = []
    for j in range(GROWS):
        descs.append(
            pltpu.async_copy(adj_hbm.at[gidx_v.at[j]], avals_v.at[j], gsem))

    @plsc.parallel_loop(0, RPT, 1, unroll=4)
    def pass1b(r):
        g = r >> 3
        o = (r & 7) << 4
        scols = iknn_v[g, pl.ds(o, 16)]
        rglob = row0 + r
        keep = jnp.zeros((16,), jnp.int32)
        for s in range(NSEEDS):
            lr = plsc.load_gather(
                labels_v, [jnp.full((16,), s * N, jnp.int32) + rglob])
            ln = plsc.load_gather(labels_v, [scols + s * N])
            keep = keep | (ln == lr).astype(jnp.int32)
        lk_v[pl.ds(r * 16, 16)] = keep

    for d in descs:
        d.wait()

    @plsc.parallel_loop(0, EPT // 16, 1, unroll=4)
    def pass2(v):
        g = v >> 3
        o = (v & 7) << 4
        av = avals_v[g, pl.ds(o, 16)]
        lk = lk_v[pl.ds(v * 16, 16)]
        k01 = ((av != 0.0) | (lk != 0)).astype(jnp.int32)
        k01_v[pl.ds(v * 16, 16)] = k01
        nkv_s[v] = jnp.sum(k01)

    def prefix(v, run):
        t = nkv_s[v]
        nkv_s[v] = run
        return run + t

    kcount = lax.fori_loop(0, EPT // 16, prefix, jnp.int32(0))

    me_v[...] = jnp.where(lanes == sid, kcount, 0)
    pltpu.sync_copy(me_v, shared.at[sid])
    plsc.subcore_barrier()
    pltpu.sync_copy(shared, call_v)
    counts = jnp.zeros((16,), jnp.int32)
    for t in range(NTILES):
        counts = counts + call_v[t, :]
    total_kept = jnp.sum(counts)
    kbase = jnp.sum(jnp.where(lanes < sid, counts, 0))
    dbase = sid * EPT - kbase

    @plsc.parallel_loop(0, EPT // 16, 1, unroll=4)
    def pass3(v):
        g = v >> 3
        o = (v & 7) << 4
        k01 = k01_v[pl.ds(v * 16, 16)]
        keep = k01 != 0
        base = nkv_s[v]
        kdest = kbase + base + plsc.cumsum(k01) - 1
        ddest = (total_kept + dbase + (v * 16 - base)
                 + plsc.cumsum(1 - k01) - 1)
        dest_v[g, pl.ds(o, 16)] = jnp.where(keep, kdest, ddest)
        rowv_v[g, pl.ds(o, 16)] = jnp.where(keep, row0 + v, 0)
        colv_v[g, pl.ds(o, 16)] = jnp.where(keep, iknn_v[g, pl.ds(o, 16)], 0)

    descs = []
    for j in range(GROWS):
        descs.append(
            pltpu.async_copy(rowv_v.at[j], rows_hbm.at[dest_v.at[j]], ssem))
        descs.append(
            pltpu.async_copy(colv_v.at[j], cols_hbm.at[dest_v.at[j]], ssem))
    for d in descs:
        d.wait()


def _sc_build(adj_flat, iknn2d, labels_flat):
    mesh = plsc.VectorSubcoreMesh(
        core_axis_name="c", subcore_axis_name="s", num_cores=1)
    f = pl.kernel(
        _sc_body,
        compiler_params=pltpu.CompilerParams(
            use_tc_tiling_on_sc=False, needs_layout_passes=False),
        out_type=(
            jax.ShapeDtypeStruct((N * K,), jnp.int32),
            jax.ShapeDtypeStruct((N * K,), jnp.int32),
        ),
        mesh=mesh,
        scratch_types=[
            pltpu.VMEM((GROWS, 128), jnp.int32),
            pltpu.VMEM((NSEEDS * N,), jnp.int32),
            pltpu.VMEM((EPT,), jnp.int32),
            pltpu.VMEM((EPT,), jnp.int32),
            pltpu.VMEM((GROWS, 128), jnp.int32),
            pltpu.VMEM((GROWS, 128), jnp.float32),
            pltpu.VMEM((GROWS, 128), jnp.int32),
            pltpu.VMEM((GROWS, 128), jnp.int32),
            pltpu.VMEM((GROWS, 128), jnp.int32),
            pltpu.VMEM((16,), jnp.int32),
            pltpu.VMEM((16, 16), jnp.int32),
            pltpu.SMEM((EPT // 16,), jnp.int32),
            pltpu.VMEM_SHARED((16, 16), jnp.int32),
            pltpu.SemaphoreType.DMA,
            pltpu.SemaphoreType.DMA,
        ],
    )
    return f(adj_flat, iknn2d, labels_flat)


def kernel(adj, student, teacher, topk):
    teacher_t = teacher.T
    i_knn = _topk(student, teacher_t)

    cent0 = []
    for s in range(NSEEDS):
        kk = jax.random.key(s + 1234)
        init_idx = jax.random.choice(kk, N, (NCENT,), replace=False)
        cent0.append(teacher[init_idx])
    cent0 = jnp.stack(cent0)
    labels = _kmeans_labels(teacher, teacher_t, cent0)

    rows_out, cols_out = _sc_build(
        adj.reshape(-1), i_knn.reshape(N * K // 128, 128),
        labels.reshape(-1))
    indices = jnp.stack([rows_out, cols_out], axis=0)
    return indices, topk


# speedup vs baseline: 16.2346x; 1.0332x over previous
import jax
import jax.numpy as jnp
from jax import lax
from jax.experimental import pallas as pl
from jax.experimental.pallas import tpu as pltpu
from jax.experimental.pallas import tpu_sc as plsc

N = 4096
D = 64
K = 16
NCENT = 32
NSEEDS = 5
NITER = 20

ROWS_BLK = 256
NBLK = N // ROWS_BLK


def _topk_body(s_ref, tT_ref, out_ref):
    i = pl.program_id(0)
    sim = lax.dot_general(
        s_ref[...], tT_ref[...], (((1,), (0,)), ((), ())),
        preferred_element_type=jnp.float32)
    rows = i * ROWS_BLK + lax.broadcasted_iota(jnp.int32, (ROWS_BLK, 1), 0)
    cols = lax.broadcasted_iota(jnp.int32, (ROWS_BLK, N), 1)
    work = jnp.where(cols == rows, sim + 10.0, sim)
    neg = jnp.float32(-jnp.inf)
    picked = []
    for _ in range(K):
        m = jnp.max(work, axis=1, keepdims=True)
        idx = jnp.min(jnp.where(work == m, cols, N), axis=1, keepdims=True)
        picked.append(idx)
        work = jnp.where(cols == idx, neg, work)
    k = 2
    while k <= K:
        j = k // 2
        while j >= 1:
            for i in range(K):
                l = i ^ j
                if l > i:
                    a, b = picked[i], picked[l]
                    lo, hi = jnp.minimum(a, b), jnp.maximum(a, b)
                    if (i & k) == 0:
                        picked[i], picked[l] = lo, hi
                    else:
                        picked[i], picked[l] = hi, lo
            j //= 2
        k *= 2
    out_ref[...] = jnp.concatenate(picked, axis=1)


def _topk(student, teacher_t):
    return pl.pallas_call(
        _topk_body,
        grid=(NBLK,),
        in_specs=[
            pl.BlockSpec((ROWS_BLK, D), lambda i: (i, 0)),
            pl.BlockSpec((D, N), lambda i: (0, 0)),
        ],
        out_specs=pl.BlockSpec((ROWS_BLK, K), lambda i: (i, 0)),
        out_shape=jax.ShapeDtypeStruct((N, K), jnp.int32),
    )(student, teacher_t)


def _kmeans_body(x_ref, xT_ref, c0_ref, out_ref):
    x = x_ref[...]
    xT = xT_ref[...]
    xsqT = jnp.sum(xT * xT, axis=0, keepdims=True)
    iota_c = lax.broadcasted_iota(jnp.int32, (NCENT, N), 0)

    def assign(c):
        csq = jnp.sum(c * c, axis=1, keepdims=True)
        prod = lax.dot_general(
            c, xT, (((1,), (0,)), ((), ())),
            preferred_element_type=jnp.float32)
        d2 = xsqT - 2.0 * prod + csq
        m = jnp.min(d2, axis=0, keepdims=True)
        return jnp.min(jnp.where(d2 == m, iota_c, NCENT), axis=0,
                       keepdims=True)

    def step(_, c):
        lbl = assign(c)
        h = (iota_c == lbl).astype(jnp.float32)
        sums = lax.dot_general(
            h, x, (((1,), (0,)), ((), ())),
            preferred_element_type=jnp.float32)
        counts = jnp.sum(h, axis=1, keepdims=True)
        return jnp.where(counts > 0.0,
                         sums / jnp.maximum(counts, 1.0), c)

    c = lax.fori_loop(0, NITER, step, c0_ref[0])
    out_ref[...] = assign(c)[None]


def _kmeans_labels(x, x_t, cent0):
    return pl.pallas_call(
        _kmeans_body,
        grid=(NSEEDS,),
        in_specs=[
            pl.BlockSpec((N, D), lambda s: (0, 0)),
            pl.BlockSpec((D, N), lambda s: (0, 0)),
            pl.BlockSpec((1, NCENT, D), lambda s: (s, 0, 0)),
        ],
        out_specs=pl.BlockSpec((1, 1, N), lambda s: (s, 0, 0)),
        out_shape=jax.ShapeDtypeStruct((NSEEDS, 1, N), jnp.int32),
    )(x, x_t, cent0)


NTILES = 16
RPT = N // NTILES
EPT = RPT * K
GROWS = EPT // 128


def _sc_body(adj_hbm, iknn_hbm, labels_hbm, rows_hbm, cols_hbm,
             iknn_v, labels_v, lk_v, k01_v,
             gidx_v, avals_v, dest_v, rowv_v, colv_v,
             me_v, call_v, nkv_s, shared, gsem, ssem):
    sid = lax.axis_index("s")
    row0 = sid * RPT
    lanes = lax.iota(jnp.int32, 16)

    pltpu.sync_copy(iknn_hbm.at[pl.ds(sid * GROWS, GROWS)], iknn_v)
    pltpu.sync_copy(labels_hbm, labels_v)

    @plsc.parallel_loop(0, RPT, 1, unroll=8)
    def pass1a(r):
        g = r >> 3
        o = (r & 7) << 4
        gidx_v[pl.ds(r * 16, 16)] = (row0 + r) * N + iknn_v[g, pl.ds(o, 16)]

    gdesc = pltpu.async_copy(adj_hbm.at[gidx_v], avals_v, gsem)

    @plsc.parallel_loop(0, RPT, 1, unroll=4)
    def pass1b(r):
        g = r >> 3
        o = (r & 7) << 4
        scols = iknn_v[g, pl.ds(o, 16)]
        rglob = row0 + r
        keep = jnp.zeros((16,), jnp.int32)
        for s in range(NSEEDS):
            lr = plsc.load_gather(
                labels_v, [jnp.full((16,), s * N, jnp.int32) + rglob])
            ln = plsc.load_gather(labels_v, [scols + s * N])
            keep = keep | (ln == lr).astype(jnp.int32)
        lk_v[pl.ds(r * 16, 16)] = keep

    gdesc.wait()

    @plsc.parallel_loop(0, EPT // 16, 1, unroll=4)
    def pass2(v):
        av = avals_v[pl.ds(v * 16, 16)]
        lk = lk_v[pl.ds(v * 16, 16)]
        k01 = ((av != 0.0) | (lk != 0)).astype(jnp.int32)
        k01_v[pl.ds(v * 16, 16)] = k01
        nkv_s[v] = jnp.sum(k01)

    def prefix(v, run):
        t = nkv_s[v]
        nkv_s[v] = run
        return run + t

    kcount = lax.fori_loop(0, EPT // 16, prefix, jnp.int32(0))

    me_v[...] = jnp.where(lanes == sid, kcount, 0)
    pltpu.sync_copy(me_v, shared.at[sid])
    plsc.subcore_barrier()
    pltpu.sync_copy(shared, call_v)
    counts = jnp.zeros((16,), jnp.int32)
    for t in range(NTILES):
        counts = counts + call_v[t, :]
    total_kept = jnp.sum(counts)
    kbase = jnp.sum(jnp.where(lanes < sid, counts, 0))
    dbase = sid * EPT - kbase

    @plsc.parallel_loop(0, EPT // 16, 1, unroll=4)
    def pass3(v):
        g = v >> 3
        o = (v & 7) << 4
        k01 = k01_v[pl.ds(v * 16, 16)]
        keep = k01 != 0
        base = nkv_s[v]
        kdest = kbase + base + plsc.cumsum(k01) - 1
        ddest = (total_kept + dbase + (v * 16 - base)
                 + plsc.cumsum(1 - k01) - 1)
        dest_v[pl.ds(v * 16, 16)] = jnp.where(keep, kdest, ddest)
        rowv_v[pl.ds(v * 16, 16)] = jnp.where(keep, row0 + v, 0)
        colv_v[pl.ds(v * 16, 16)] = jnp.where(keep, iknn_v[g, pl.ds(o, 16)], 0)

    d1 = pltpu.async_copy(rowv_v, rows_hbm.at[dest_v], ssem)
    d2 = pltpu.async_copy(colv_v, cols_hbm.at[dest_v], ssem)
    d1.wait()
    d2.wait()


def _sc_build(adj_flat, iknn2d, labels_flat):
    mesh = plsc.VectorSubcoreMesh(
        core_axis_name="c", subcore_axis_name="s", num_cores=1)
    f = pl.kernel(
        _sc_body,
        compiler_params=pltpu.CompilerParams(
            use_tc_tiling_on_sc=False, needs_layout_passes=False),
        out_type=(
            jax.ShapeDtypeStruct((N * K,), jnp.int32),
            jax.ShapeDtypeStruct((N * K,), jnp.int32),
        ),
        mesh=mesh,
        scratch_types=[
            pltpu.VMEM((GROWS, 128), jnp.int32),
            pltpu.VMEM((NSEEDS * N,), jnp.int32),
            pltpu.VMEM((EPT,), jnp.int32),
            pltpu.VMEM((EPT,), jnp.int32),
            pltpu.VMEM((EPT,), jnp.int32),
            pltpu.VMEM((EPT,), jnp.float32),
            pltpu.VMEM((EPT,), jnp.int32),
            pltpu.VMEM((EPT,), jnp.int32),
            pltpu.VMEM((EPT,), jnp.int32),
            pltpu.VMEM((16,), jnp.int32),
            pltpu.VMEM((16, 16), jnp.int32),
            pltpu.SMEM((EPT // 16,), jnp.int32),
            pltpu.VMEM_SHARED((16, 16), jnp.int32),
            pltpu.SemaphoreType.DMA,
            pltpu.SemaphoreType.DMA,
        ],
    )
    return f(adj_flat, iknn2d, labels_flat)


def kernel(adj, student, teacher, topk):
    teacher_t = teacher.T
    i_knn = _topk(student, teacher_t)

    cent0 = []
    for s in range(NSEEDS):
        kk = jax.random.key(s + 1234)
        init_idx = jax.random.choice(kk, N, (NCENT,), replace=False)
        cent0.append(teacher[init_idx])
    cent0 = jnp.stack(cent0)
    labels = _kmeans_labels(teacher, teacher_t, cent0)

    rows_out, cols_out = _sc_build(
        adj.reshape(-1), i_knn.reshape(N * K // 128, 128),
        labels.reshape(-1))
    indices = jnp.stack([rows_out, cols_out], axis=0)
    return indices, topk
